# baseline (speedup 1.0000x reference)
import jax
import jax.numpy as jnp
from jax.experimental import pallas as pl
from jax.experimental.pallas import tpu as pltpu

N = 1706
NP = 1792
H = 8
GH = 64
HD = H * GH
GA = 128
FFN = 264
FFNP = 384
THR = 0.98
BN = 448
NBLK = NP // BN
CHUNKS = ((0, 512), (512, 512), (1024, 512), (1536, 256))
F32 = jnp.float32
BF16 = jnp.bfloat16


def _leaky(x):
    return jnp.where(x >= 0, x, 0.2 * x)


def _sigmoid(x):
    e = jnp.exp(-jnp.abs(x))
    return jnp.where(x >= 0, 1.0 / (1.0 + e), e / (1.0 + e))


def _elu(x):
    return jnp.where(x > 0, x, jnp.exp(jnp.minimum(x, 0.0)) - 1.0)


def _ones_col(n):
    lanes = jax.lax.broadcasted_iota(jnp.int32, (n, GA - GH), 1)
    return jnp.where(lanes == 0, 1.0, 0.0).astype(BF16)


def _write_heads(r, h2b_ref, h2a_ref):
    h2b_ref[...] = r.astype(BF16)
    for h in range(H):
        h2a_ref[h, :, 0:GH] = r[:, h * GH:(h + 1) * GH].astype(BF16)
        h2a_ref[h, :, GH:GA] = _ones_col(BN)


def _proj_kernel(x_ref, w_ref, h2b_ref, h2a_ref):
    pid = pl.program_id(0)
    colmask = jax.lax.broadcasted_iota(jnp.int32, (1, NP), 1) < N
    rowmask = (jax.lax.broadcasted_iota(jnp.int32, (BN, 1), 0) + pid * BN) < N
    x = jnp.where(colmask & rowmask, x_ref[...], 0.0).astype(BF16)
    _write_heads(jnp.dot(x, w_ref[...], preferred_element_type=F32),
                 h2b_ref, h2a_ref)


def _attention_rows(pid, adj_ref, h2b_ref, h2a_ref, asrc_ref, adst_ref,
                    keepb_ref):
    row0 = pid * BN
    h2rows = h2b_ref[pl.ds(row0, BN), :]
    est = jnp.dot(h2rows, asrc_ref[...],
                  preferred_element_type=F32)
    edt_rows = jnp.dot(h2rows, adst_ref[...],
                       preferred_element_type=F32)
    edt = jnp.dot(h2b_ref[...], adst_ref[...],
                  preferred_element_type=F32)
    maxd_row = jnp.max(edt, axis=0, keepdims=True)
    ed = jnp.transpose(edt)
    maxd_col = jnp.max(ed, axis=1, keepdims=True)
    edm = ed - maxd_col
    b_all = jnp.exp(edm)
    d_all = jnp.exp(0.2 * edm)
    xm = est + maxd_row
    shift = _leaky(xm)
    a_all = jnp.exp(xm - shift)
    c_all = jnp.exp(0.2 * xm - shift)

    rows_glob = jax.lax.broadcasted_iota(jnp.int32, (BN, 1), 0) + row0
    diag = jnp.zeros((BN, 1), F32)
    for c0, bm in CHUNKS:
        a = adj_ref[:, c0:c0 + bm]
        keep = a > THR
        if c0 + bm > N:
            keep = keep & (jax.lax.broadcasted_iota(jnp.int32, (1, bm), 1)
                           < (N - c0))
        keepb_ref[:, c0:c0 + bm] = jnp.where(keep, 1.0, 0.0).astype(BF16)
        cols = jax.lax.broadcasted_iota(jnp.int32, (1, bm), 1) + c0
        diag = diag + jnp.sum(jnp.where(cols == rows_glob, a, 0.0),
                              axis=1, keepdims=True)

    rows_real = rows_glob < N
    p_self = jnp.exp(_leaky(est + edt_rows) - shift)
    s_self = jnp.where((diag > THR) & rows_real, 0.0, p_self)

    bb = b_all.astype(BF16)
    dd = d_all.astype(BF16)
    aa = a_all.astype(BF16)
    cc = c_all.astype(BF16)
    emb = []
    for h in range(H):
        a_pos = aa[:, h:h + 1]
        c_neg = cc[:, h:h + 1]
        h2aug_rows = h2a_ref[h, pl.ds(row0, BN), :].astype(F32)
        acc = s_self[:, h:h + 1] * h2aug_rows
        for c0, bm in CHUNKS:
            pv = jnp.maximum(bb[h:h + 1, c0:c0 + bm] * a_pos,
                             dd[h:h + 1, c0:c0 + bm] * c_neg)
            p = pv * keepb_ref[:, c0:c0 + bm]
            acc = acc + jnp.dot(p, h2a_ref[h, pl.ds(c0, bm), :],
                                preferred_element_type=F32)
        emb.append(_elu(acc[:, 0:GH] * (1.0 / acc[:, GH:GH + 1])))
    return jnp.concatenate(emb, axis=1)


def _ffn_sig(embcat, wg_ref, bg_ref, w1_ref, b1_ref, w2_ref, b2_ref):
    t = jnp.dot(embcat.astype(BF16), wg_ref[...],
                preferred_element_type=F32) + bg_ref[...]
    u = jnp.dot(jnp.maximum(t, 0.0).astype(BF16), w1_ref[...],
                preferred_element_type=F32) + b1_ref[...]
    logits = jnp.dot(jnp.maximum(u, 0.0).astype(BF16), w2_ref[...],
                     preferred_element_type=F32) + b2_ref[...]
    return _sigmoid(logits)


def _gat_ffn_next_kernel(adj_ref, h2b_ref, h2a_ref, asrc_ref, adst_ref,
                         wg_ref, bg_ref, w1_ref, b1_ref, w2_ref, b2_ref,
                         wn_ref, out_ref, h2nb_ref, h2na_ref, keepb_ref):
    pid = pl.program_id(0)
    embcat = _attention_rows(pid, adj_ref, h2b_ref, h2a_ref, asrc_ref,
                             adst_ref, keepb_ref)
    sig = _ffn_sig(embcat, wg_ref, bg_ref, w1_ref, b1_ref, w2_ref, b2_ref)
    out_ref[...] = sig
    _write_heads(jnp.dot(sig.astype(BF16), wn_ref[...],
                         preferred_element_type=F32), h2nb_ref, h2na_ref)


def _gat_ffn_kernel(adj_ref, h2b_ref, h2a_ref, asrc_ref, adst_ref,
                    wg_ref, bg_ref, w1_ref, b1_ref, w2_ref, b2_ref, out_ref,
                    keepb_ref):
    pid = pl.program_id(0)
    embcat = _attention_rows(pid, adj_ref, h2b_ref, h2a_ref, asrc_ref,
                             adst_ref, keepb_ref)
    out_ref[...] = _ffn_sig(embcat, wg_ref, bg_ref, w1_ref, b1_ref, w2_ref,
                            b2_ref)


def _pad2(x, r, c):
    return jnp.pad(x, ((0, r - x.shape[0]), (0, c - x.shape[1])))


def _full(shape):
    nd = len(shape)
    return pl.BlockSpec(shape, lambda i: (0,) * nd)


def _block_diag(a):
    eye = jnp.eye(H, dtype=a.dtype)
    return (a[:, :, None] * eye[:, None, :]).reshape(HD, H)


def _flat_w(w):
    return jnp.pad(jnp.transpose(w, (1, 0, 2)).reshape(N, HD),
                   ((0, NP - N), (0, 0))).astype(BF16)


def kernel(batch, ddi_adj, sim_adj, W_ddi, a_src_ddi, a_dst_ddi, Wg1, bg1,
           W11, b11, W12, b12, W_sim, a_src_sim, a_dst_sim, Wg2, bg2, W21,
           b21, W22, b22):
    wddi = _flat_w(W_ddi)
    wsim = _flat_w(W_sim)
    asrc1 = _block_diag(a_src_ddi).astype(BF16)
    adst1 = _block_diag(a_dst_ddi).astype(BF16)
    asrc2 = _block_diag(a_src_sim).astype(BF16)
    adst2 = _block_diag(a_dst_sim).astype(BF16)
    wg1 = _pad2(Wg1, HD, FFNP).astype(BF16)
    wg2 = _pad2(Wg2, HD, FFNP).astype(BF16)
    w11 = _pad2(W11, FFNP, FFNP).astype(BF16)
    w21 = _pad2(W21, FFNP, FFNP).astype(BF16)
    w12 = _pad2(W12, FFNP, NP).astype(BF16)
    w22 = _pad2(W22, FFNP, NP).astype(BF16)
    bg1p = _pad2(bg1[None, :], 1, FFNP)
    bg2p = _pad2(bg2[None, :], 1, FFNP)
    b11p = _pad2(b11[None, :], 1, FFNP)
    b21p = _pad2(b21[None, :], 1, FFNP)
    b12p = _pad2(b12[None, :], 1, NP)
    b22p = _pad2(b22[None, :], 1, NP)

    h2b_ddi, h2a_ddi = pl.pallas_call(
        _proj_kernel,
        grid=(NBLK,),
        in_specs=[pl.BlockSpec((BN, NP), lambda i: (i, 0)),
                  _full((NP, HD))],
        out_specs=[pl.BlockSpec((BN, HD), lambda i: (i, 0)),
                   pl.BlockSpec((H, BN, GA), lambda i: (0, i, 0))],
        out_shape=[jax.ShapeDtypeStruct((NP, HD), BF16),
                   jax.ShapeDtypeStruct((H, NP, GA), BF16)],
    )(batch, wddi)

    out1, h2b_sim, h2a_sim = pl.pallas_call(
        _gat_ffn_next_kernel,
        grid=(NBLK,),
        in_specs=[pl.BlockSpec((BN, NP), lambda i: (i, 0)),
                  _full((NP, HD)), _full((H, NP, GA)),
                  _full((HD, H)), _full((HD, H)),
                  _full((HD, FFNP)), _full((1, FFNP)),
                  _full((FFNP, FFNP)), _full((1, FFNP)),
                  _full((FFNP, NP)), _full((1, NP)),
                  _full((NP, HD))],
        out_specs=[pl.BlockSpec((BN, NP), lambda i: (i, 0)),
                   pl.BlockSpec((BN, HD), lambda i: (i, 0)),
                   pl.BlockSpec((H, BN, GA), lambda i: (0, i, 0))],
        out_shape=[jax.ShapeDtypeStruct((N, N), F32),
                   jax.ShapeDtypeStruct((NP, HD), BF16),
                   jax.ShapeDtypeStruct((H, NP, GA), BF16)],
        scratch_shapes=[pltpu.VMEM((BN, NP), BF16)],
    )(ddi_adj, h2b_ddi, h2a_ddi, asrc1, adst1, wg1, bg1p,
      w11, b11p, w12, b12p, wsim)

    out2 = pl.pallas_call(
        _gat_ffn_kernel,
        grid=(NBLK,),
        in_specs=[pl.BlockSpec((BN, NP), lambda i: (i, 0)),
                  _full((NP, HD)), _full((H, NP, GA)),
                  _full((HD, H)), _full((HD, H)),
                  _full((HD, FFNP)), _full((1, FFNP)),
                  _full((FFNP, FFNP)), _full((1, FFNP)),
                  _full((FFNP, NP)), _full((1, NP))],
        out_specs=pl.BlockSpec((BN, NP), lambda i: (i, 0)),
        out_shape=jax.ShapeDtypeStruct((N, N), F32),
        scratch_shapes=[pltpu.VMEM((BN, NP), BF16)],
    )(sim_adj, h2b_sim, h2a_sim, asrc2, adst2, wg2, bg2p,
      w21, b21p, w22, b22p)

    return (out1, out2)


# speedup vs baseline: 1.0815x; 1.0815x over previous
import jax
import jax.numpy as jnp
from jax.experimental import pallas as pl
from jax.experimental.pallas import tpu as pltpu

N = 1706
NP = 1792
H = 8
GH = 64
HD = H * GH
GA = 128
FFN = 264
FFNP = 384
THR = 0.98
BN = 896
NBLK = NP // BN
CHUNKS = ((0, 512), (512, 512), (1024, 512), (1536, 256))
F32 = jnp.float32
BF16 = jnp.bfloat16


def _leaky(x):
    return jnp.where(x >= 0, x, 0.2 * x)


def _sigmoid(x):
    e = jnp.exp(-jnp.abs(x))
    return jnp.where(x >= 0, 1.0 / (1.0 + e), e / (1.0 + e))


def _elu(x):
    return jnp.where(x > 0, x, jnp.exp(jnp.minimum(x, 0.0)) - 1.0)


def _ones_col(n):
    lanes = jax.lax.broadcasted_iota(jnp.int32, (n, GA - GH), 1)
    return jnp.where(lanes == 0, 1.0, 0.0).astype(BF16)


def _write_heads(r, h2b_ref, h2a_ref):
    h2b_ref[...] = r.astype(BF16)
    for h in range(H):
        h2a_ref[h, :, 0:GH] = r[:, h * GH:(h + 1) * GH].astype(BF16)
        h2a_ref[h, :, GH:GA] = _ones_col(BN)


def _proj_kernel(x_ref, w_ref, h2b_ref, h2a_ref):
    pid = pl.program_id(0)
    colmask = jax.lax.broadcasted_iota(jnp.int32, (1, NP), 1) < N
    rowmask = (jax.lax.broadcasted_iota(jnp.int32, (BN, 1), 0) + pid * BN) < N
    x = jnp.where(colmask & rowmask, x_ref[...], 0.0).astype(BF16)
    _write_heads(jnp.dot(x, w_ref[...], preferred_element_type=F32),
                 h2b_ref, h2a_ref)


def _attention_rows(pid, adj_ref, h2b_ref, h2a_ref, asrc_ref, adst_ref,
                    keepb_ref):
    row0 = pid * BN
    h2rows = h2b_ref[pl.ds(row0, BN), :]
    est = jnp.dot(h2rows, asrc_ref[...],
                  preferred_element_type=F32)
    edt_rows = jnp.dot(h2rows, adst_ref[...],
                       preferred_element_type=F32)
    edt = jnp.dot(h2b_ref[...], adst_ref[...],
                  preferred_element_type=F32)
    maxd_row = jnp.max(edt, axis=0, keepdims=True)
    ed = jnp.transpose(edt)
    maxd_col = jnp.max(ed, axis=1, keepdims=True)
    edm = ed - maxd_col
    b_all = jnp.exp(edm)
    d_all = jnp.exp(0.2 * edm)
    xm = est + maxd_row
    shift = _leaky(xm)
    a_all = jnp.exp(xm - shift)
    c_all = jnp.exp(0.2 * xm - shift)

    rows_glob = jax.lax.broadcasted_iota(jnp.int32, (BN, 1), 0) + row0
    diag = jnp.zeros((BN, 1), F32)
    for c0, bm in CHUNKS:
        a = adj_ref[:, c0:c0 + bm]
        keep = a > THR
        if c0 + bm > N:
            keep = keep & (jax.lax.broadcasted_iota(jnp.int32, (1, bm), 1)
                           < (N - c0))
        keepb_ref[:, c0:c0 + bm] = jnp.where(keep, 1.0, 0.0).astype(BF16)
        cols = jax.lax.broadcasted_iota(jnp.int32, (1, bm), 1) + c0
        diag = diag + jnp.sum(jnp.where(cols == rows_glob, a, 0.0),
                              axis=1, keepdims=True)

    rows_real = rows_glob < N
    p_self = jnp.exp(_leaky(est + edt_rows) - shift)
    s_self = jnp.where((diag > THR) & rows_real, 0.0, p_self)

    bb = b_all.astype(BF16)
    dd = d_all.astype(BF16)
    aa = a_all.astype(BF16)
    cc = c_all.astype(BF16)
    emb = []
    for h in range(H):
        a_pos = aa[:, h:h + 1]
        c_neg = cc[:, h:h + 1]
        h2aug_rows = h2a_ref[h, pl.ds(row0, BN), :].astype(F32)
        acc = s_self[:, h:h + 1] * h2aug_rows
        for c0, bm in CHUNKS:
            pv = jnp.maximum(bb[h:h + 1, c0:c0 + bm] * a_pos,
                             dd[h:h + 1, c0:c0 + bm] * c_neg)
            p = pv * keepb_ref[:, c0:c0 + bm]
            acc = acc + jnp.dot(p, h2a_ref[h, pl.ds(c0, bm), :],
                                preferred_element_type=F32)
        emb.append(_elu(acc[:, 0:GH] * (1.0 / acc[:, GH:GH + 1])))
    return jnp.concatenate(emb, axis=1)


def _ffn_sig(embcat, wg_ref, bg_ref, w1_ref, b1_ref, w2_ref, b2_ref):
    t = jnp.dot(embcat.astype(BF16), wg_ref[...],
                preferred_element_type=F32) + bg_ref[...]
    u = jnp.dot(jnp.maximum(t, 0.0).astype(BF16), w1_ref[...],
                preferred_element_type=F32) + b1_ref[...]
    logits = jnp.dot(jnp.maximum(u, 0.0).astype(BF16), w2_ref[...],
                     preferred_element_type=F32) + b2_ref[...]
    return _sigmoid(logits)


def _gat_ffn_next_kernel(adj_ref, h2b_ref, h2a_ref, asrc_ref, adst_ref,
                         wg_ref, bg_ref, w1_ref, b1_ref, w2_ref, b2_ref,
                         wn_ref, out_ref, h2nb_ref, h2na_ref, keepb_ref):
    pid = pl.program_id(0)
    embcat = _attention_rows(pid, adj_ref, h2b_ref, h2a_ref, asrc_ref,
                             adst_ref, keepb_ref)
    sig = _ffn_sig(embcat, wg_ref, bg_ref, w1_ref, b1_ref, w2_ref, b2_ref)
    out_ref[...] = sig
    _write_heads(jnp.dot(sig.astype(BF16), wn_ref[...],
                         preferred_element_type=F32), h2nb_ref, h2na_ref)


def _gat_ffn_kernel(adj_ref, h2b_ref, h2a_ref, asrc_ref, adst_ref,
                    wg_ref, bg_ref, w1_ref, b1_ref, w2_ref, b2_ref, out_ref,
                    keepb_ref):
    pid = pl.program_id(0)
    embcat = _attention_rows(pid, adj_ref, h2b_ref, h2a_ref, asrc_ref,
                             adst_ref, keepb_ref)
    out_ref[...] = _ffn_sig(embcat, wg_ref, bg_ref, w1_ref, b1_ref, w2_ref,
                            b2_ref)


def _pad2(x, r, c):
    return jnp.pad(x, ((0, r - x.shape[0]), (0, c - x.shape[1])))


def _full(shape):
    nd = len(shape)
    return pl.BlockSpec(shape, lambda i: (0,) * nd)


def _block_diag(a):
    eye = jnp.eye(H, dtype=a.dtype)
    return (a[:, :, None] * eye[:, None, :]).reshape(HD, H)


def _flat_w(w):
    return jnp.pad(jnp.transpose(w, (1, 0, 2)).reshape(N, HD),
                   ((0, NP - N), (0, 0))).astype(BF16)


def kernel(batch, ddi_adj, sim_adj, W_ddi, a_src_ddi, a_dst_ddi, Wg1, bg1,
           W11, b11, W12, b12, W_sim, a_src_sim, a_dst_sim, Wg2, bg2, W21,
           b21, W22, b22):
    wddi = _flat_w(W_ddi)
    wsim = _flat_w(W_sim)
    asrc1 = _block_diag(a_src_ddi).astype(BF16)
    adst1 = _block_diag(a_dst_ddi).astype(BF16)
    asrc2 = _block_diag(a_src_sim).astype(BF16)
    adst2 = _block_diag(a_dst_sim).astype(BF16)
    wg1 = _pad2(Wg1, HD, FFNP).astype(BF16)
    wg2 = _pad2(Wg2, HD, FFNP).astype(BF16)
    w11 = _pad2(W11, FFNP, FFNP).astype(BF16)
    w21 = _pad2(W21, FFNP, FFNP).astype(BF16)
    w12 = _pad2(W12, FFNP, NP).astype(BF16)
    w22 = _pad2(W22, FFNP, NP).astype(BF16)
    bg1p = _pad2(bg1[None, :], 1, FFNP)
    bg2p = _pad2(bg2[None, :], 1, FFNP)
    b11p = _pad2(b11[None, :], 1, FFNP)
    b21p = _pad2(b21[None, :], 1, FFNP)
    b12p = _pad2(b12[None, :], 1, NP)
    b22p = _pad2(b22[None, :], 1, NP)

    h2b_ddi, h2a_ddi = pl.pallas_call(
        _proj_kernel,
        grid=(NBLK,),
        in_specs=[pl.BlockSpec((BN, NP), lambda i: (i, 0)),
                  _full((NP, HD))],
        out_specs=[pl.BlockSpec((BN, HD), lambda i: (i, 0)),
                   pl.BlockSpec((H, BN, GA), lambda i: (0, i, 0))],
        out_shape=[jax.ShapeDtypeStruct((NP, HD), BF16),
                   jax.ShapeDtypeStruct((H, NP, GA), BF16)],
    )(batch, wddi)

    out1, h2b_sim, h2a_sim = pl.pallas_call(
        _gat_ffn_next_kernel,
        grid=(NBLK,),
        in_specs=[pl.BlockSpec((BN, NP), lambda i: (i, 0)),
                  _full((NP, HD)), _full((H, NP, GA)),
                  _full((HD, H)), _full((HD, H)),
                  _full((HD, FFNP)), _full((1, FFNP)),
                  _full((FFNP, FFNP)), _full((1, FFNP)),
                  _full((FFNP, NP)), _full((1, NP)),
                  _full((NP, HD))],
        out_specs=[pl.BlockSpec((BN, NP), lambda i: (i, 0)),
                   pl.BlockSpec((BN, HD), lambda i: (i, 0)),
                   pl.BlockSpec((H, BN, GA), lambda i: (0, i, 0))],
        out_shape=[jax.ShapeDtypeStruct((N, N), F32),
                   jax.ShapeDtypeStruct((NP, HD), BF16),
                   jax.ShapeDtypeStruct((H, NP, GA), BF16)],
        scratch_shapes=[pltpu.VMEM((BN, NP), BF16)],
    )(ddi_adj, h2b_ddi, h2a_ddi, asrc1, adst1, wg1, bg1p,
      w11, b11p, w12, b12p, wsim)

    out2 = pl.pallas_call(
        _gat_ffn_kernel,
        grid=(NBLK,),
        in_specs=[pl.BlockSpec((BN, NP), lambda i: (i, 0)),
                  _full((NP, HD)), _full((H, NP, GA)),
                  _full((HD, H)), _full((HD, H)),
                  _full((HD, FFNP)), _full((1, FFNP)),
                  _full((FFNP, FFNP)), _full((1, FFNP)),
                  _full((FFNP, NP)), _full((1, NP))],
        out_specs=pl.BlockSpec((BN, NP), lambda i: (i, 0)),
        out_shape=jax.ShapeDtypeStruct((N, N), F32),
        scratch_shapes=[pltpu.VMEM((BN, NP), BF16)],
    )(sim_adj, h2b_sim, h2a_sim, asrc2, adst2, wg2, bg2p,
      w21, b21p, w22, b22p)

    return (out1, out2)


# speedup vs baseline: 1.1046x; 1.0214x over previous
import jax
import jax.numpy as jnp
from jax.experimental import pallas as pl
from jax.experimental.pallas import tpu as pltpu

N = 1706
NP = 1792
H = 8
GH = 64
HD = H * GH
GA = 128
FFN = 264
FFNP = 384
THR = 0.98
BN = 896
NBLK = NP // BN
CHUNKS = ((0, 512), (512, 512), (1024, 512), (1536, 256))
F32 = jnp.float32
BF16 = jnp.bfloat16


def _leaky(x):
    return jnp.where(x >= 0, x, 0.2 * x)


def _sigmoid(x):
    e = jnp.exp(-jnp.abs(x))
    return jnp.where(x >= 0, 1.0 / (1.0 + e), e / (1.0 + e))


def _elu(x):
    return jnp.where(x > 0, x, jnp.exp(jnp.minimum(x, 0.0)) - 1.0)


def _ones_col(n):
    lanes = jax.lax.broadcasted_iota(jnp.int32, (n, GA - GH), 1)
    return jnp.where(lanes == 0, 1.0, 0.0).astype(BF16)


def _write_heads(r, h2b_ref, h2a_ref):
    h2b_ref[...] = r.astype(BF16)
    for h in range(H):
        h2a_ref[h, :, 0:GH] = r[:, h * GH:(h + 1) * GH].astype(BF16)
        h2a_ref[h, :, GH:GA] = _ones_col(BN)


def _proj_kernel(x_ref, w_ref, h2b_ref, h2a_ref):
    pid = pl.program_id(0)
    colmask = jax.lax.broadcasted_iota(jnp.int32, (1, NP), 1) < N
    rowmask = (jax.lax.broadcasted_iota(jnp.int32, (BN, 1), 0) + pid * BN) < N
    x = jnp.where(colmask & rowmask, x_ref[...], 0.0).astype(BF16)
    _write_heads(jnp.dot(x, w_ref[...], preferred_element_type=F32),
                 h2b_ref, h2a_ref)


def _attention_rows(pid, adj_ref, h2b_ref, h2a_ref, asrc_ref, adst_ref,
                    keepb_ref):
    row0 = pid * BN
    h2rows = h2b_ref[pl.ds(row0, BN), :]
    est = jnp.dot(h2rows, asrc_ref[...],
                  preferred_element_type=F32)
    edt_rows = jnp.dot(h2rows, adst_ref[...],
                       preferred_element_type=F32)
    edt = jnp.dot(h2b_ref[...], adst_ref[...],
                  preferred_element_type=F32)
    maxd_row = jnp.max(edt, axis=0, keepdims=True)
    ed = jnp.transpose(edt)
    maxd_col = jnp.max(ed, axis=1, keepdims=True)
    edm = ed - maxd_col
    b_all = jnp.exp(edm)
    d_all = jnp.exp(0.2 * edm)
    xm = est + maxd_row
    shift = _leaky(xm)
    a_all = jnp.exp(xm - shift)
    c_all = jnp.exp(0.2 * xm - shift)

    ri = jax.lax.broadcasted_iota(jnp.int32, (BN, BN), 0)
    ci = jax.lax.broadcasted_iota(jnp.int32, (BN, BN), 1)
    dblk = adj_ref[:, pl.ds(row0, BN)]
    diag = jnp.sum(jnp.where(ri == ci, dblk, 0.0), axis=1,
                   keepdims=True)
    rows_real = (jax.lax.broadcasted_iota(jnp.int32, (BN, 1), 0) + row0) < N
    p_self = jnp.exp(_leaky(est + edt_rows) - shift)
    s_self = jnp.where((diag > THR) & rows_real, 0.0, p_self)

    for c0, bm in CHUNKS:
        keep = adj_ref[:, c0:c0 + bm] > THR
        if c0 + bm > N:
            keep = keep & (jax.lax.broadcasted_iota(jnp.int32, (1, bm), 1)
                           < (N - c0))
        keepb_ref[:, c0:c0 + bm] = jnp.where(keep, 1.0, 0.0).astype(BF16)

    bb = b_all.astype(BF16)
    dd = d_all.astype(BF16)
    aa = a_all.astype(BF16)
    cc = c_all.astype(BF16)
    emb = []
    for h in range(H):
        a_pos = aa[:, h:h + 1]
        c_neg = cc[:, h:h + 1]
        h2aug_rows = h2a_ref[h, pl.ds(row0, BN), :].astype(F32)
        acc = s_self[:, h:h + 1] * h2aug_rows
        for c0, bm in CHUNKS:
            pv = jnp.maximum(bb[h:h + 1, c0:c0 + bm] * a_pos,
                             dd[h:h + 1, c0:c0 + bm] * c_neg)
            p = pv * keepb_ref[:, c0:c0 + bm]
            acc = acc + jnp.dot(p, h2a_ref[h, pl.ds(c0, bm), :],
                                preferred_element_type=F32)
        emb.append(_elu(acc[:, 0:GH] * (1.0 / acc[:, GH:GH + 1])))
    return jnp.concatenate(emb, axis=1)


def _ffn_sig(embcat, wg_ref, bg_ref, w1_ref, b1_ref, w2_ref, b2_ref):
    t = jnp.dot(embcat.astype(BF16), wg_ref[...],
                preferred_element_type=F32) + bg_ref[...]
    u = jnp.dot(jnp.maximum(t, 0.0).astype(BF16), w1_ref[...],
                preferred_element_type=F32) + b1_ref[...]
    logits = jnp.dot(jnp.maximum(u, 0.0).astype(BF16), w2_ref[...],
                     preferred_element_type=F32) + b2_ref[...]
    return _sigmoid(logits)


def _gat_ffn_next_kernel(adj_ref, h2b_ref, h2a_ref, asrc_ref, adst_ref,
                         wg_ref, bg_ref, w1_ref, b1_ref, w2_ref, b2_ref,
                         wn_ref, out_ref, h2nb_ref, h2na_ref, keepb_ref):
    pid = pl.program_id(0)
    embcat = _attention_rows(pid, adj_ref, h2b_ref, h2a_ref, asrc_ref,
                             adst_ref, keepb_ref)
    sig = _ffn_sig(embcat, wg_ref, bg_ref, w1_ref, b1_ref, w2_ref, b2_ref)
    out_ref[...] = sig
    _write_heads(jnp.dot(sig.astype(BF16), wn_ref[...],
                         preferred_element_type=F32), h2nb_ref, h2na_ref)


def _gat_ffn_kernel(adj_ref, h2b_ref, h2a_ref, asrc_ref, adst_ref,
                    wg_ref, bg_ref, w1_ref, b1_ref, w2_ref, b2_ref, out_ref,
                    keepb_ref):
    pid = pl.program_id(0)
    embcat = _attention_rows(pid, adj_ref, h2b_ref, h2a_ref, asrc_ref,
                             adst_ref, keepb_ref)
    out_ref[...] = _ffn_sig(embcat, wg_ref, bg_ref, w1_ref, b1_ref, w2_ref,
                            b2_ref)


def _pad2(x, r, c):
    return jnp.pad(x, ((0, r - x.shape[0]), (0, c - x.shape[1])))


def _full(shape):
    nd = len(shape)
    return pl.BlockSpec(shape, lambda i: (0,) * nd)


def _block_diag(a):
    eye = jnp.eye(H, dtype=a.dtype)
    return (a[:, :, None] * eye[:, None, :]).reshape(HD, H)


def _flat_w(w):
    return jnp.pad(jnp.transpose(w, (1, 0, 2)).reshape(N, HD),
                   ((0, NP - N), (0, 0))).astype(BF16)


def kernel(batch, ddi_adj, sim_adj, W_ddi, a_src_ddi, a_dst_ddi, Wg1, bg1,
           W11, b11, W12, b12, W_sim, a_src_sim, a_dst_sim, Wg2, bg2, W21,
           b21, W22, b22):
    wddi = _flat_w(W_ddi)
    wsim = _flat_w(W_sim)
    asrc1 = _block_diag(a_src_ddi).astype(BF16)
    adst1 = _block_diag(a_dst_ddi).astype(BF16)
    asrc2 = _block_diag(a_src_sim).astype(BF16)
    adst2 = _block_diag(a_dst_sim).astype(BF16)
    wg1 = _pad2(Wg1, HD, FFNP).astype(BF16)
    wg2 = _pad2(Wg2, HD, FFNP).astype(BF16)
    w11 = _pad2(W11, FFNP, FFNP).astype(BF16)
    w21 = _pad2(W21, FFNP, FFNP).astype(BF16)
    w12 = _pad2(W12, FFNP, NP).astype(BF16)
    w22 = _pad2(W22, FFNP, NP).astype(BF16)
    bg1p = _pad2(bg1[None, :], 1, FFNP)
    bg2p = _pad2(bg2[None, :], 1, FFNP)
    b11p = _pad2(b11[None, :], 1, FFNP)
    b21p = _pad2(b21[None, :], 1, FFNP)
    b12p = _pad2(b12[None, :], 1, NP)
    b22p = _pad2(b22[None, :], 1, NP)

    h2b_ddi, h2a_ddi = pl.pallas_call(
        _proj_kernel,
        grid=(NBLK,),
        in_specs=[pl.BlockSpec((BN, NP), lambda i: (i, 0)),
                  _full((NP, HD))],
        out_specs=[pl.BlockSpec((BN, HD), lambda i: (i, 0)),
                   pl.BlockSpec((H, BN, GA), lambda i: (0, i, 0))],
        out_shape=[jax.ShapeDtypeStruct((NP, HD), BF16),
                   jax.ShapeDtypeStruct((H, NP, GA), BF16)],
    )(batch, wddi)

    out1, h2b_sim, h2a_sim = pl.pallas_call(
        _gat_ffn_next_kernel,
        grid=(NBLK,),
        in_specs=[pl.BlockSpec((BN, NP), lambda i: (i, 0)),
                  _full((NP, HD)), _full((H, NP, GA)),
                  _full((HD, H)), _full((HD, H)),
                  _full((HD, FFNP)), _full((1, FFNP)),
                  _full((FFNP, FFNP)), _full((1, FFNP)),
                  _full((FFNP, NP)), _full((1, NP)),
                  _full((NP, HD))],
        out_specs=[pl.BlockSpec((BN, NP), lambda i: (i, 0)),
                   pl.BlockSpec((BN, HD), lambda i: (i, 0)),
                   pl.BlockSpec((H, BN, GA), lambda i: (0, i, 0))],
        out_shape=[jax.ShapeDtypeStruct((N, N), F32),
                   jax.ShapeDtypeStruct((NP, HD), BF16),
                   jax.ShapeDtypeStruct((H, NP, GA), BF16)],
        scratch_shapes=[pltpu.VMEM((BN, NP), BF16)],
    )(ddi_adj, h2b_ddi, h2a_ddi, asrc1, adst1, wg1, bg1p,
      w11, b11p, w12, b12p, wsim)

    out2 = pl.pallas_call(
        _gat_ffn_kernel,
        grid=(NBLK,),
        in_specs=[pl.BlockSpec((BN, NP), lambda i: (i, 0)),
                  _full((NP, HD)), _full((H, NP, GA)),
                  _full((HD, H)), _full((HD, H)),
                  _full((HD, FFNP)), _full((1, FFNP)),
                  _full((FFNP, FFNP)), _full((1, FFNP)),
                  _full((FFNP, NP)), _full((1, NP))],
        out_specs=pl.BlockSpec((BN, NP), lambda i: (i, 0)),
        out_shape=jax.ShapeDtypeStruct((N, N), F32),
        scratch_shapes=[pltpu.VMEM((BN, NP), BF16)],
    )(sim_adj, h2b_sim, h2a_sim, asrc2, adst2, wg2, bg2p,
      w21, b21p, w22, b22p)

    return (out1, out2)
